# Initial kernel scaffold; baseline (speedup 1.0000x reference)
import jax
import jax.numpy as jnp
from jax.experimental import pallas as pl


def kernel(x, pos, edge_index, batch, params):
    raise NotImplementedError("write your pallas kernel here")


# speedup vs baseline: 2.8115x; 2.8115x over previous
import functools

import jax
import jax.numpy as jnp
from jax import lax
from jax.experimental import pallas as pl
from jax.experimental.pallas import tpu as pltpu
from jax.experimental.pallas import tpu_sc as plsc

H = 128
NNODE = 10000
NEDGE = 320000
NGRAPH = 64
NPAD = 10240
GPAD = 128
NC, NS, LANE = 2, 16, 16
NW = NC * NS

F32 = jnp.float32


def _mesh():
    return plsc.VectorSubcoreMesh(
        core_axis_name="c", subcore_axis_name="s", num_cores=NC, num_subcores=NS
    )


def _sc_d2(pos_x, pos_y, pos_z, dst, src):
    EPW = NEDGE // NW

    @functools.partial(
        pl.kernel,
        out_type=jax.ShapeDtypeStruct((NEDGE,), F32),
        mesh=_mesh(),
        scratch_types=[
            pltpu.VMEM((NNODE,), F32),
            pltpu.VMEM((NNODE,), F32),
            pltpu.VMEM((NNODE,), F32),
            pltpu.VMEM((EPW,), jnp.int32),
            pltpu.VMEM((EPW,), jnp.int32),
            pltpu.VMEM((EPW,), F32),
        ],
        compiler_params=pltpu.CompilerParams(needs_layout_passes=False),
    )
    def k(px_h, py_h, pz_h, dst_h, src_h, out_h, px, py, pz, dv, sv, ov):
        wid = lax.axis_index("c") * NS + lax.axis_index("s")
        base = pl.multiple_of(wid * EPW, 8)
        pltpu.sync_copy(px_h, px)
        pltpu.sync_copy(py_h, py)
        pltpu.sync_copy(pz_h, pz)
        pltpu.sync_copy(dst_h.at[pl.ds(base, EPW)], dv)
        pltpu.sync_copy(src_h.at[pl.ds(base, EPW)], sv)

        def body(i, carry):
            o = i * LANE
            di = dv[pl.ds(o, LANE)]
            si = sv[pl.ds(o, LANE)]
            rx = plsc.load_gather(px, [di]) - plsc.load_gather(px, [si])
            ry = plsc.load_gather(py, [di]) - plsc.load_gather(py, [si])
            rz = plsc.load_gather(pz, [di]) - plsc.load_gather(pz, [si])
            ov[pl.ds(o, LANE)] = rx * rx + ry * ry + rz * rz
            return carry

        lax.fori_loop(0, EPW // LANE, body, 0)
        pltpu.sync_copy(ov, out_h.at[pl.ds(base, EPW)])

    return k(pos_x, pos_y, pos_z, dst, src)


def _sc_gather(table, idx):
    rows = idx.shape[0]
    nchunks = rows // 128
    workers = 25 if nchunks % 32 else 32
    ncpw = nchunks // workers

    @functools.partial(
        pl.kernel,
        out_type=jax.ShapeDtypeStruct((rows, H), F32),
        mesh=_mesh(),
        scratch_types=[
            pltpu.VMEM((128,), jnp.int32),
            pltpu.VMEM((128, H), F32),
            pltpu.SemaphoreType.DMA,
        ],
    )
    def k(tab_h, idx_h, out_h, idx_v, rows_v, sem):
        wid = lax.axis_index("c") * NS + lax.axis_index("s")

        @pl.when(wid < workers)
        def _():
            base = pl.multiple_of(wid * (ncpw * 128), 8)

            def body(j, carry):
                ofs = pl.multiple_of(base + j * 128, 8)
                pltpu.sync_copy(idx_h.at[pl.ds(ofs, 128)], idx_v)
                pltpu.async_copy(tab_h.at[idx_v], rows_v, sem).wait()
                pltpu.sync_copy(rows_v, out_h.at[pl.ds(ofs, 128)])
                return carry

            lax.fori_loop(0, ncpw, body, 0)

    return k(table, idx)


def _sc_scatter(rows_arr, idx, ntable, chunk):
    nrows = rows_arr.shape[0]
    nchunks = nrows // chunk
    workers = 25 if nchunks % 32 else 32
    ncpw = nchunks // workers
    assert nchunks == workers * ncpw and nrows == nchunks * chunk
    assert ntable % (NS * 8) == 0
    zrows = ntable // NS

    @functools.partial(
        pl.kernel,
        out_type=jax.ShapeDtypeStruct((2, ntable, H), F32),
        mesh=_mesh(),
        scratch_types=[
            pltpu.VMEM((chunk,), jnp.int32),
            pltpu.VMEM((chunk, H), F32),
            pltpu.VMEM_SHARED((ntable, H), F32),
        ],
    )
    def k(rows_h, idx_h, out_h, idx_v, rows_v, shared):
        c = lax.axis_index("c")
        s = lax.axis_index("s")
        wid = c * NS + s

        zv = jnp.zeros((LANE,), F32)

        def zb(r, carry):
            for cc in range(H // LANE):
                rows_v[r, pl.ds(cc * LANE, LANE)] = zv
            return carry

        lax.fori_loop(0, chunk, zb, 0)
        zoff = 0
        rem = zrows
        while rem > 0:
            n = min(chunk, rem)
            pltpu.sync_copy(
                rows_v.at[pl.ds(0, n)], shared.at[pl.ds(s * zrows + zoff, n)]
            )
            zoff += n
            rem -= n
        plsc.subcore_barrier()

        @pl.when(wid < workers)
        def _():
            base = pl.multiple_of(wid * (ncpw * chunk), 8)

            def body(j, carry):
                ofs = pl.multiple_of(base + j * chunk, 8)
                pltpu.sync_copy(idx_h.at[pl.ds(ofs, chunk)], idx_v)
                pltpu.sync_copy(rows_h.at[pl.ds(ofs, chunk)], rows_v)
                pltpu.sync_copy(rows_v, shared.at[idx_v], add=True)
                return carry

            lax.fori_loop(0, ncpw, body, 0)

        plsc.subcore_barrier()
        pltpu.sync_copy(
            shared.at[pl.ds(s * zrows, zrows)], out_h.at[c, pl.ds(s * zrows, zrows)]
        )

    return k(rows_arr, idx)


def _ln(x, g, b):
    m = jnp.mean(x, axis=-1, keepdims=True)
    v = jnp.mean((x - m) ** 2, axis=-1, keepdims=True)
    return (x - m) * lax.rsqrt(v + 1e-5) * g + b


def _silu(x):
    return x * jax.nn.sigmoid(x)


def _full(shape):
    return pl.BlockSpec(shape, lambda i: tuple(0 for _ in shape))


def _tc_embed(x, W_emb, b_emb, Wd, Ws, b1):
    BN = 400
    grid = NNODE // BN

    def body(x_r, We_r, be_r, Wd_r, Ws_r, b1_r, h_r, T_r):
        h = jnp.dot(x_r[...], We_r[...], preferred_element_type=F32) + be_r[...]
        h_r[...] = h
        T_r[0] = jnp.dot(h, Wd_r[...], preferred_element_type=F32) + b1_r[...]
        T_r[1] = jnp.dot(h, Ws_r[...], preferred_element_type=F32)

    return pl.pallas_call(
        body,
        grid=(grid,),
        in_specs=[
            pl.BlockSpec((BN, H), lambda i: (i, 0)),
            _full((H, H)),
            _full((1, H)),
            _full((H, H)),
            _full((H, H)),
            _full((1, H)),
        ],
        out_specs=[
            pl.BlockSpec((BN, H), lambda i: (i, 0)),
            pl.BlockSpec((2, BN, H), lambda i: (0, i, 0)),
        ],
        out_shape=[
            jax.ShapeDtypeStruct((NNODE, H), F32),
            jax.ShapeDtypeStruct((2, NNODE, H), F32),
        ],
    )(x, W_emb, b_emb.reshape(1, H), Wd, Ws, b1.reshape(1, H))


def _tc_edge(GG3, d2c, w1d, g1, be1, W2, b2, g2, be2):
    BE = 1280
    grid = NEDGE // BE

    def body(gg_r, d2_r, wd_r, g1_r, be1_r, W2_r, b2_r, g2_r, be2_r, out_r):
        a = gg_r[0]
        b = gg_r[1]
        dist = jnp.sqrt(d2_r[...] + 1e-12)
        pre = a + b + dist * wd_r[...]
        t = _silu(_ln(pre, g1_r[...], be1_r[...]))
        t2 = jnp.dot(t, W2_r[...], preferred_element_type=F32) + b2_r[...]
        out_r[...] = _silu(_ln(t2, g2_r[...], be2_r[...]))

    return pl.pallas_call(
        body,
        grid=(grid,),
        in_specs=[
            pl.BlockSpec((2, BE, H), lambda i: (0, i, 0)),
            pl.BlockSpec((BE, 1), lambda i: (i, 0)),
            _full((1, H)),
            _full((1, H)),
            _full((1, H)),
            _full((H, H)),
            _full((1, H)),
            _full((1, H)),
            _full((1, H)),
        ],
        out_specs=pl.BlockSpec((BE, H), lambda i: (i, 0)),
        out_shape=jax.ShapeDtypeStruct((NEDGE, H), F32),
    )(GG3, d2c, w1d, g1.reshape(1, H), be1.reshape(1, H), W2,
      b2.reshape(1, H), g2.reshape(1, H), be2.reshape(1, H))


def _tc_node(h, P, up, nxt):
    BN = 400
    grid = NNODE // BN
    has_next = nxt is not None

    def body(h_r, p_r, W1h_r, W1a_r, b1_r, g1_r, be1_r, W2_r, b2_r, g2_r,
             be2_r, *rest):
        if has_next:
            Wd_r, Ws_r, bn_r, h_o, T_o = rest
        else:
            (h_o,) = rest
        h0 = h_r[...]
        agg = p_r[0] + p_r[1]
        u = jnp.dot(h0, W1h_r[...], preferred_element_type=F32)
        u = u + jnp.dot(agg, W1a_r[...], preferred_element_type=F32) + b1_r[...]
        u = _silu(_ln(u, g1_r[...], be1_r[...]))
        u = jnp.dot(u, W2_r[...], preferred_element_type=F32) + b2_r[...]
        u = _silu(_ln(u, g2_r[...], be2_r[...]))
        hn = h0 + u
        h_o[...] = hn
        if has_next:
            T_o[0] = jnp.dot(hn, Wd_r[...], preferred_element_type=F32) + bn_r[...]
            T_o[1] = jnp.dot(hn, Ws_r[...], preferred_element_type=F32)

    in_specs = [
        pl.BlockSpec((BN, H), lambda i: (i, 0)),
        pl.BlockSpec((2, BN, H), lambda i: (0, i, 0)),
        _full((H, H)),
        _full((H, H)),
        _full((1, H)),
        _full((1, H)),
        _full((1, H)),
        _full((H, H)),
        _full((1, H)),
        _full((1, H)),
        _full((1, H)),
    ]
    args = [h, P, up["W1"][:H], up["W1"][H:], up["b1"].reshape(1, H),
            up["g1"].reshape(1, H), up["be1"].reshape(1, H), up["W2"],
            up["b2"].reshape(1, H), up["g2"].reshape(1, H),
            up["be2"].reshape(1, H)]
    out_specs = [pl.BlockSpec((BN, H), lambda i: (i, 0))]
    out_shape = [jax.ShapeDtypeStruct((NNODE, H), F32)]
    if has_next:
        Wd, Ws, bn = nxt
        in_specs += [_full((H, H)), _full((H, H)), _full((1, H))]
        args += [Wd, Ws, bn.reshape(1, H)]
        out_specs.append(pl.BlockSpec((2, BN, H), lambda i: (0, i, 0)))
        out_shape.append(jax.ShapeDtypeStruct((2, NNODE, H), F32))

    res = pl.pallas_call(
        body,
        grid=(grid,),
        in_specs=in_specs,
        out_specs=out_specs,
        out_shape=out_shape,
    )(*args)
    return res if has_next else (res[0], None)


def _tc_head(Q, Wp1, bp1, Wp2, bp2):
    def body(q_r, W1_r, b1_r, W2_r, b2_r, out_r):
        q = q_r[0] + q_r[1]
        o = jnp.maximum(jnp.dot(q, W1_r[...], preferred_element_type=F32)
                        + b1_r[...], 0.0)
        out_r[...] = jnp.dot(o, W2_r[...], preferred_element_type=F32) + b2_r[...]

    return pl.pallas_call(
        body,
        grid=(1,),
        in_specs=[
            pl.BlockSpec((2, NGRAPH, H), lambda i: (0, 0, 0)),
            _full((H, H)),
            _full((1, H)),
            _full((H, 1)),
            _full((1, 1)),
        ],
        out_specs=_full((NGRAPH, 1)),
        out_shape=jax.ShapeDtypeStruct((NGRAPH, 1), F32),
    )(Q, Wp1, bp1.reshape(1, H), Wp2, bp2.reshape(1, 1))


def kernel(x, pos, edge_index, batch, params):
    src = edge_index[0].astype(jnp.int32)
    dst = edge_index[1].astype(jnp.int32)
    idx2 = jnp.concatenate([dst, src + NNODE])
    pos_x = pos[:, 0]
    pos_y = pos[:, 1]
    pos_z = pos[:, 2]

    d2 = _sc_d2(pos_x, pos_y, pos_z, dst, src)
    d2c = d2.reshape(NEDGE, 1)

    layers = params["layers"]

    def msg_split(l):
        W1 = layers[l]["msg"]["W1"]
        return W1[:H], W1[H : 2 * H], W1[2 * H :], layers[l]["msg"]["b1"]

    Wd0, Ws0, _, b10 = msg_split(0)
    h, T = _tc_embed(x, params["W_emb"], params["b_emb"], Wd0, Ws0, b10)

    for l in range(len(layers)):
        mp = layers[l]["msg"]
        _, _, w1d, _ = msg_split(l)
        GG = _sc_gather(T.reshape(2 * NNODE, H), idx2)
        msg = _tc_edge(GG.reshape(2, NEDGE, H), d2c, w1d, mp["g1"], mp["be1"],
                       mp["W2"], mp["b2"], mp["g2"], mp["be2"])
        P = _sc_scatter(msg, dst, NPAD, 128)
        if l + 1 < len(layers):
            Wd, Ws, _, b1n = msg_split(l + 1)
            h, T = _tc_node(h, P, layers[l]["upd"], (Wd, Ws, b1n))
        else:
            h, _ = _tc_node(h, P, layers[l]["upd"], None)

    Q = _sc_scatter(h, batch.astype(jnp.int32), GPAD, 80)
    return _tc_head(Q, params["Wp1"], params["bp1"], params["Wp2"], params["bp2"])


# speedup vs baseline: 3.7640x; 1.3388x over previous
import functools

import jax
import jax.numpy as jnp
from jax import lax
from jax.experimental import pallas as pl
from jax.experimental.pallas import tpu as pltpu
from jax.experimental.pallas import tpu_sc as plsc

H = 128
NNODE = 10000
NEDGE = 320000
NGRAPH = 64
NPAD = 10240
GPAD = 128
NC, NS, LANE = 2, 16, 16
NW = NC * NS

F32 = jnp.float32


def _mesh():
    return plsc.VectorSubcoreMesh(
        core_axis_name="c", subcore_axis_name="s", num_cores=NC, num_subcores=NS
    )


def _sc_d2(pos_x, pos_y, pos_z, dst, src):
    EPW = NEDGE // NW

    @functools.partial(
        pl.kernel,
        out_type=jax.ShapeDtypeStruct((NEDGE,), F32),
        mesh=_mesh(),
        scratch_types=[
            pltpu.VMEM((NNODE,), F32),
            pltpu.VMEM((NNODE,), F32),
            pltpu.VMEM((NNODE,), F32),
            pltpu.VMEM((EPW,), jnp.int32),
            pltpu.VMEM((EPW,), jnp.int32),
            pltpu.VMEM((EPW,), F32),
        ],
        compiler_params=pltpu.CompilerParams(needs_layout_passes=False),
    )
    def k(px_h, py_h, pz_h, dst_h, src_h, out_h, px, py, pz, dv, sv, ov):
        wid = lax.axis_index("c") * NS + lax.axis_index("s")
        base = pl.multiple_of(wid * EPW, 8)
        pltpu.sync_copy(px_h, px)
        pltpu.sync_copy(py_h, py)
        pltpu.sync_copy(pz_h, pz)
        pltpu.sync_copy(dst_h.at[pl.ds(base, EPW)], dv)
        pltpu.sync_copy(src_h.at[pl.ds(base, EPW)], sv)

        def body(i, carry):
            o = i * LANE
            di = dv[pl.ds(o, LANE)]
            si = sv[pl.ds(o, LANE)]
            rx = plsc.load_gather(px, [di]) - plsc.load_gather(px, [si])
            ry = plsc.load_gather(py, [di]) - plsc.load_gather(py, [si])
            rz = plsc.load_gather(pz, [di]) - plsc.load_gather(pz, [si])
            ov[pl.ds(o, LANE)] = rx * rx + ry * ry + rz * rz
            return carry

        lax.fori_loop(0, EPW // LANE, body, 0)
        pltpu.sync_copy(ov, out_h.at[pl.ds(base, EPW)])

    return k(pos_x, pos_y, pos_z, dst, src)


def _sc_gather(table, idx):
    rows = idx.shape[0]
    nchunks = rows // 128
    workers = 25 if nchunks % 32 else 32
    ncpw = nchunks // workers

    assert ncpw % 2 == 0
    npairs = ncpw // 2

    @functools.partial(
        pl.kernel,
        out_type=jax.ShapeDtypeStruct((rows, H), F32),
        mesh=_mesh(),
        scratch_types=[
            pltpu.VMEM((2, 128), jnp.int32),
            pltpu.VMEM((2, 128, H), F32),
            pltpu.SemaphoreType.DMA,
            pltpu.SemaphoreType.DMA,
            pltpu.SemaphoreType.DMA,
            pltpu.SemaphoreType.DMA,
            pltpu.SemaphoreType.DMA,
            pltpu.SemaphoreType.DMA,
        ],
    )
    def k(tab_h, idx_h, out_h, idx_v, rows_v, si0, si1, sg0, sg1, ss0, ss1):
        wid = lax.axis_index("c") * NS + lax.axis_index("s")

        @pl.when(wid < workers)
        def _():
            base = pl.multiple_of(wid * (ncpw * 128), 8)
            si = (si0, si1)
            sg = (sg0, sg1)
            ss = (ss0, ss1)

            def idx_copy(j, b):
                ofs = pl.multiple_of(base + j * 128, 8)
                return pltpu.make_async_copy(
                    idx_h.at[pl.ds(ofs, 128)], idx_v.at[b], si[b]
                )

            def gath(b):
                return pltpu.make_async_copy(
                    tab_h.at[idx_v.at[b]], rows_v.at[b], sg[b]
                )

            def store(j, b):
                ofs = pl.multiple_of(base + j * 128, 8)
                return pltpu.make_async_copy(
                    rows_v.at[b], out_h.at[pl.ds(ofs, 128)], ss[b]
                )

            idx_copy(0, 0).start()

            def body(i, carry):
                j0 = i * 2
                j1 = j0 + 1
                idx_copy(j0, 0).wait()

                @pl.when(i > 0)
                def _():
                    store(j0 - 2, 0).wait()

                gath(0).start()
                idx_copy(j1, 1).start()
                gath(0).wait()
                store(j0, 0).start()
                idx_copy(j1, 1).wait()

                @pl.when(i > 0)
                def _():
                    store(j1 - 2, 1).wait()

                gath(1).start()

                @pl.when(i + 1 < npairs)
                def _():
                    idx_copy(j0 + 2, 0).start()

                gath(1).wait()
                store(j1, 1).start()
                return carry

            lax.fori_loop(0, npairs, body, 0)
            store(ncpw - 2, 0).wait()
            store(ncpw - 1, 1).wait()

    return k(table, idx)


def _sc_scatter(rows_arr, idx, ntable, chunk):
    nrows = rows_arr.shape[0]
    nchunks = nrows // chunk
    workers = 25 if nchunks % 32 else 32
    ncpw = nchunks // workers
    assert nchunks == workers * ncpw and nrows == nchunks * chunk
    assert ntable % (NS * 8) == 0
    zrows = ntable // NS

    @functools.partial(
        pl.kernel,
        out_type=jax.ShapeDtypeStruct((2, ntable, H), F32),
        mesh=_mesh(),
        scratch_types=[
            pltpu.VMEM((2, chunk), jnp.int32),
            pltpu.VMEM((2, chunk, H), F32),
            pltpu.VMEM_SHARED((ntable, H), F32),
            pltpu.SemaphoreType.DMA,
            pltpu.SemaphoreType.DMA,
            pltpu.SemaphoreType.DMA,
            pltpu.SemaphoreType.DMA,
        ],
    )
    def k(rows_h, idx_h, out_h, idx_v, rows_v, shared, si0, si1, sr0, sr1):
        c = lax.axis_index("c")
        s = lax.axis_index("s")
        wid = c * NS + s

        zv = jnp.zeros((LANE,), F32)

        def zb(r, carry):
            for cc in range(H // LANE):
                rows_v[0, r, pl.ds(cc * LANE, LANE)] = zv
            return carry

        lax.fori_loop(0, chunk, zb, 0)
        zoff = 0
        rem = zrows
        while rem > 0:
            n = min(chunk, rem)
            pltpu.sync_copy(
                rows_v.at[0, pl.ds(0, n)], shared.at[pl.ds(s * zrows + zoff, n)]
            )
            zoff += n
            rem -= n
        plsc.subcore_barrier()

        @pl.when(wid < workers)
        def _():
            base = pl.multiple_of(wid * (ncpw * chunk), 8)
            si = (si0, si1)
            sr = (sr0, sr1)

            def idx_copy(j, b):
                ofs = pl.multiple_of(base + j * chunk, 8)
                return pltpu.make_async_copy(
                    idx_h.at[pl.ds(ofs, chunk)], idx_v.at[b], si[b]
                )

            def rows_copy(j, b):
                ofs = pl.multiple_of(base + j * chunk, 8)
                return pltpu.make_async_copy(
                    rows_h.at[pl.ds(ofs, chunk)], rows_v.at[b], sr[b]
                )

            def scat(b):
                pltpu.sync_copy(rows_v.at[b], shared.at[idx_v.at[b]], add=True)

            if ncpw % 2 == 0:
                npairs = ncpw // 2
                idx_copy(0, 0).start()
                rows_copy(0, 0).start()

                def body(i, carry):
                    j0 = i * 2
                    j1 = j0 + 1
                    idx_copy(j1, 1).start()
                    rows_copy(j1, 1).start()
                    idx_copy(j0, 0).wait()
                    rows_copy(j0, 0).wait()
                    scat(0)

                    @pl.when(i + 1 < npairs)
                    def _():
                        idx_copy(j0 + 2, 0).start()
                        rows_copy(j0 + 2, 0).start()

                    idx_copy(j1, 1).wait()
                    rows_copy(j1, 1).wait()
                    scat(1)
                    return carry

                lax.fori_loop(0, npairs, body, 0)
            else:

                def body(j, carry):
                    ofs = pl.multiple_of(base + j * chunk, 8)
                    pltpu.sync_copy(idx_h.at[pl.ds(ofs, chunk)], idx_v.at[0])
                    pltpu.sync_copy(rows_h.at[pl.ds(ofs, chunk)], rows_v.at[0])
                    scat(0)
                    return carry

                lax.fori_loop(0, ncpw, body, 0)

        plsc.subcore_barrier()
        pltpu.sync_copy(
            shared.at[pl.ds(s * zrows, zrows)], out_h.at[c, pl.ds(s * zrows, zrows)]
        )

    return k(rows_arr, idx)


def _ln(x, g, b):
    m = jnp.mean(x, axis=-1, keepdims=True)
    v = jnp.mean((x - m) ** 2, axis=-1, keepdims=True)
    return (x - m) * lax.rsqrt(v + 1e-5) * g + b


def _silu(x):
    return x * jax.nn.sigmoid(x)


def _full(shape):
    return pl.BlockSpec(shape, lambda i: tuple(0 for _ in shape))


def _tc_embed(x, W_emb, b_emb, Wd, Ws, b1):
    BN = 400
    grid = NNODE // BN

    def body(x_r, We_r, be_r, Wd_r, Ws_r, b1_r, h_r, T_r):
        h = jnp.dot(x_r[...], We_r[...], preferred_element_type=F32) + be_r[...]
        h_r[...] = h
        T_r[0] = jnp.dot(h, Wd_r[...], preferred_element_type=F32) + b1_r[...]
        T_r[1] = jnp.dot(h, Ws_r[...], preferred_element_type=F32)

    return pl.pallas_call(
        body,
        grid=(grid,),
        in_specs=[
            pl.BlockSpec((BN, H), lambda i: (i, 0)),
            _full((H, H)),
            _full((1, H)),
            _full((H, H)),
            _full((H, H)),
            _full((1, H)),
        ],
        out_specs=[
            pl.BlockSpec((BN, H), lambda i: (i, 0)),
            pl.BlockSpec((2, BN, H), lambda i: (0, i, 0)),
        ],
        out_shape=[
            jax.ShapeDtypeStruct((NNODE, H), F32),
            jax.ShapeDtypeStruct((2, NNODE, H), F32),
        ],
    )(x, W_emb, b_emb.reshape(1, H), Wd, Ws, b1.reshape(1, H))


def _tc_edge(GG3, d2c, w1d, g1, be1, W2, b2, g2, be2):
    BE = 1280
    grid = NEDGE // BE

    def body(gg_r, d2_r, wd_r, g1_r, be1_r, W2_r, b2_r, g2_r, be2_r, out_r):
        a = gg_r[0]
        b = gg_r[1]
        dist = jnp.sqrt(d2_r[...] + 1e-12)
        pre = a + b + dist * wd_r[...]
        t = _silu(_ln(pre, g1_r[...], be1_r[...]))
        t2 = jnp.dot(t, W2_r[...], preferred_element_type=F32) + b2_r[...]
        out_r[...] = _silu(_ln(t2, g2_r[...], be2_r[...]))

    return pl.pallas_call(
        body,
        grid=(grid,),
        in_specs=[
            pl.BlockSpec((2, BE, H), lambda i: (0, i, 0)),
            pl.BlockSpec((BE, 1), lambda i: (i, 0)),
            _full((1, H)),
            _full((1, H)),
            _full((1, H)),
            _full((H, H)),
            _full((1, H)),
            _full((1, H)),
            _full((1, H)),
        ],
        out_specs=pl.BlockSpec((BE, H), lambda i: (i, 0)),
        out_shape=jax.ShapeDtypeStruct((NEDGE, H), F32),
    )(GG3, d2c, w1d, g1.reshape(1, H), be1.reshape(1, H), W2,
      b2.reshape(1, H), g2.reshape(1, H), be2.reshape(1, H))


def _tc_node(h, P, up, nxt):
    BN = 400
    grid = NNODE // BN
    has_next = nxt is not None

    def body(h_r, p_r, W1h_r, W1a_r, b1_r, g1_r, be1_r, W2_r, b2_r, g2_r,
             be2_r, *rest):
        if has_next:
            Wd_r, Ws_r, bn_r, h_o, T_o = rest
        else:
            (h_o,) = rest
        h0 = h_r[...]
        agg = p_r[0] + p_r[1]
        u = jnp.dot(h0, W1h_r[...], preferred_element_type=F32)
        u = u + jnp.dot(agg, W1a_r[...], preferred_element_type=F32) + b1_r[...]
        u = _silu(_ln(u, g1_r[...], be1_r[...]))
        u = jnp.dot(u, W2_r[...], preferred_element_type=F32) + b2_r[...]
        u = _silu(_ln(u, g2_r[...], be2_r[...]))
        hn = h0 + u
        h_o[...] = hn
        if has_next:
            T_o[0] = jnp.dot(hn, Wd_r[...], preferred_element_type=F32) + bn_r[...]
            T_o[1] = jnp.dot(hn, Ws_r[...], preferred_element_type=F32)

    in_specs = [
        pl.BlockSpec((BN, H), lambda i: (i, 0)),
        pl.BlockSpec((2, BN, H), lambda i: (0, i, 0)),
        _full((H, H)),
        _full((H, H)),
        _full((1, H)),
        _full((1, H)),
        _full((1, H)),
        _full((H, H)),
        _full((1, H)),
        _full((1, H)),
        _full((1, H)),
    ]
    args = [h, P, up["W1"][:H], up["W1"][H:], up["b1"].reshape(1, H),
            up["g1"].reshape(1, H), up["be1"].reshape(1, H), up["W2"],
            up["b2"].reshape(1, H), up["g2"].reshape(1, H),
            up["be2"].reshape(1, H)]
    out_specs = [pl.BlockSpec((BN, H), lambda i: (i, 0))]
    out_shape = [jax.ShapeDtypeStruct((NNODE, H), F32)]
    if has_next:
        Wd, Ws, bn = nxt
        in_specs += [_full((H, H)), _full((H, H)), _full((1, H))]
        args += [Wd, Ws, bn.reshape(1, H)]
        out_specs.append(pl.BlockSpec((2, BN, H), lambda i: (0, i, 0)))
        out_shape.append(jax.ShapeDtypeStruct((2, NNODE, H), F32))

    res = pl.pallas_call(
        body,
        grid=(grid,),
        in_specs=in_specs,
        out_specs=out_specs,
        out_shape=out_shape,
    )(*args)
    return res if has_next else (res[0], None)


def _tc_head(Q, Wp1, bp1, Wp2, bp2):
    def body(q_r, W1_r, b1_r, W2_r, b2_r, out_r):
        q = q_r[0] + q_r[1]
        o = jnp.maximum(jnp.dot(q, W1_r[...], preferred_element_type=F32)
                        + b1_r[...], 0.0)
        out_r[...] = jnp.dot(o, W2_r[...], preferred_element_type=F32) + b2_r[...]

    return pl.pallas_call(
        body,
        grid=(1,),
        in_specs=[
            pl.BlockSpec((2, NGRAPH, H), lambda i: (0, 0, 0)),
            _full((H, H)),
            _full((1, H)),
            _full((H, 1)),
            _full((1, 1)),
        ],
        out_specs=_full((NGRAPH, 1)),
        out_shape=jax.ShapeDtypeStruct((NGRAPH, 1), F32),
    )(Q, Wp1, bp1.reshape(1, H), Wp2, bp2.reshape(1, 1))


def kernel(x, pos, edge_index, batch, params):
    src = edge_index[0].astype(jnp.int32)
    dst = edge_index[1].astype(jnp.int32)
    idx2 = jnp.concatenate([dst, src + NNODE])
    pos_x = pos[:, 0]
    pos_y = pos[:, 1]
    pos_z = pos[:, 2]

    d2 = _sc_d2(pos_x, pos_y, pos_z, dst, src)
    d2c = d2.reshape(NEDGE, 1)

    layers = params["layers"]

    def msg_split(l):
        W1 = layers[l]["msg"]["W1"]
        return W1[:H], W1[H : 2 * H], W1[2 * H :], layers[l]["msg"]["b1"]

    Wd0, Ws0, _, b10 = msg_split(0)
    h, T = _tc_embed(x, params["W_emb"], params["b_emb"], Wd0, Ws0, b10)

    for l in range(len(layers)):
        mp = layers[l]["msg"]
        _, _, w1d, _ = msg_split(l)
        GG = _sc_gather(T.reshape(2 * NNODE, H), idx2)
        msg = _tc_edge(GG.reshape(2, NEDGE, H), d2c, w1d, mp["g1"], mp["be1"],
                       mp["W2"], mp["b2"], mp["g2"], mp["be2"])
        P = _sc_scatter(msg, dst, NPAD, 128)
        if l + 1 < len(layers):
            Wd, Ws, _, b1n = msg_split(l + 1)
            h, T = _tc_node(h, P, layers[l]["upd"], (Wd, Ws, b1n))
        else:
            h, _ = _tc_node(h, P, layers[l]["upd"], None)

    Q = _sc_scatter(h, batch.astype(jnp.int32), GPAD, 80)
    return _tc_head(Q, params["Wp1"], params["bp1"], params["Wp2"], params["bp2"])
